# Initial kernel scaffold; baseline (speedup 1.0000x reference)
import jax
import jax.numpy as jnp
from jax.experimental import pallas as pl


def kernel(x, edge_index, batch_mapping, temperature, mean, std, W_g0, b_g0, W_g1, b_g1, W_g2, b_g2, W_g3, b_g3, W_d0, b_d0, W_d1, b_d1, W_d2, b_d2, W_d3, b_d3, W_a, b_a):
    raise NotImplementedError("write your pallas kernel here")


# speedup vs baseline: 10.6590x; 10.6590x over previous
import functools

import jax
import jax.numpy as jnp
from jax import lax
from jax.experimental import pallas as pl
from jax.experimental.pallas import tpu as pltpu
from jax.experimental.pallas import tpu_sc as plsc

N = 10000
E = 320000
G = 256

NC = 2
NS = 16
NW = NC * NS
CHUNK = 125
NCHUNK_W = E // NW // CHUNK
NCHUNK_S = E // NS // CHUNK
RPAD = 10240
RPW = RPAD // NS
DEG_W = 128

D1 = 128
D2 = 256

PCW = 128
PCN = RPAD // NS // PCW
GPT = G // NS

_SC_MESH = plsc.VectorSubcoreMesh(core_axis_name="c", subcore_axis_name="s")


@functools.partial(
    pl.kernel,
    mesh=_SC_MESH,
    out_type=jax.ShapeDtypeStruct((NC, RPAD, D1), jnp.float32),
    scratch_types=[
        pltpu.VMEM((CHUNK,), jnp.int32),
        pltpu.VMEM((CHUNK,), jnp.int32),
        pltpu.VMEM((CHUNK, D1), jnp.float32),
        pltpu.VMEM_SHARED((RPAD, D1), jnp.float32),
    ],
)
def _edge_scatter_128(g_hbm, src_hbm, dst_hbm, zeros_hbm, out_hbm,
                      src_v, dst_v, rows_v, acc):
  c = lax.axis_index("c")
  s = lax.axis_index("s")
  wid = c * NS + s
  pltpu.sync_copy(zeros_hbm.at[pl.ds(s * RPW, RPW)], acc.at[pl.ds(s * RPW, RPW)])
  plsc.subcore_barrier()

  def body(j, carry):
    pltpu.sync_copy(src_hbm.at[wid, j], src_v)
    pltpu.sync_copy(dst_hbm.at[wid, j], dst_v)
    pltpu.sync_copy(g_hbm.at[src_v], rows_v)
    pltpu.sync_copy(rows_v, acc.at[dst_v], add=True)
    return carry

  lax.fori_loop(0, NCHUNK_W, body, 0)
  plsc.subcore_barrier()
  pltpu.sync_copy(acc.at[pl.ds(s * RPW, RPW)], out_hbm.at[c, pl.ds(s * RPW, RPW)])


@functools.partial(
    pl.kernel,
    mesh=_SC_MESH,
    out_type=jax.ShapeDtypeStruct((RPAD, D2), jnp.float32),
    scratch_types=[
        pltpu.VMEM((CHUNK,), jnp.int32),
        pltpu.VMEM((CHUNK,), jnp.int32),
        pltpu.VMEM((CHUNK, D1), jnp.float32),
        pltpu.VMEM_SHARED((RPAD, D1), jnp.float32),
    ],
)
def _edge_scatter_256(g_hbm, src_hbm, dst_hbm, zeros_hbm, out_hbm,
                      src_v, dst_v, rows_v, acc):
  c = lax.axis_index("c")
  s = lax.axis_index("s")
  col = pl.multiple_of(c * D1, D1)
  pltpu.sync_copy(zeros_hbm.at[pl.ds(s * RPW, RPW)], acc.at[pl.ds(s * RPW, RPW)])
  plsc.subcore_barrier()

  def body(j, carry):
    pltpu.sync_copy(src_hbm.at[s, j], src_v)
    pltpu.sync_copy(dst_hbm.at[s, j], dst_v)
    pltpu.sync_copy(g_hbm.at[src_v, pl.ds(col, D1)], rows_v)
    pltpu.sync_copy(rows_v, acc.at[dst_v], add=True)
    return carry

  lax.fori_loop(0, NCHUNK_S, body, 0)
  plsc.subcore_barrier()
  pltpu.sync_copy(acc.at[pl.ds(s * RPW, RPW)],
                  out_hbm.at[pl.ds(s * RPW, RPW), pl.ds(col, D1)])


@functools.partial(
    pl.kernel,
    mesh=_SC_MESH,
    out_type=jax.ShapeDtypeStruct((NC, RPAD, DEG_W), jnp.float32),
    scratch_types=[
        pltpu.VMEM((CHUNK,), jnp.int32),
        pltpu.VMEM((CHUNK, DEG_W), jnp.float32),
        pltpu.VMEM_SHARED((RPAD, DEG_W), jnp.float32),
    ],
)
def _degree_hist(dst_hbm, ones_hbm, zeros_hbm, out_hbm, dst_v, ones_v, acc):
  c = lax.axis_index("c")
  s = lax.axis_index("s")
  wid = c * NS + s
  pltpu.sync_copy(ones_hbm, ones_v)
  pltpu.sync_copy(zeros_hbm.at[pl.ds(s * RPW, RPW)], acc.at[pl.ds(s * RPW, RPW)])
  plsc.subcore_barrier()

  def body(j, carry):
    pltpu.sync_copy(dst_hbm.at[wid, j], dst_v)
    pltpu.sync_copy(ones_v, acc.at[dst_v], add=True)
    return carry

  lax.fori_loop(0, NCHUNK_W, body, 0)
  plsc.subcore_barrier()
  pltpu.sync_copy(acc.at[pl.ds(s * RPW, RPW)], out_hbm.at[c, pl.ds(s * RPW, RPW)])


@functools.partial(
    pl.kernel,
    mesh=_SC_MESH,
    out_type=jax.ShapeDtypeStruct((G, D2), jnp.float32),
    scratch_types=[
        pltpu.VMEM((PCW,), jnp.int32),
        pltpu.VMEM((PCW, D1), jnp.float32),
        pltpu.VMEM_SHARED((G, D1), jnp.float32),
    ],
)
def _pool_scatter(x_hbm, bm_hbm, zeros_hbm, out_hbm, idx_v, rows_v, acc):
  c = lax.axis_index("c")
  s = lax.axis_index("s")
  col = pl.multiple_of(c * D1, D1)
  pltpu.sync_copy(zeros_hbm.at[pl.ds(s * GPT, GPT)], acc.at[pl.ds(s * GPT, GPT)])
  plsc.subcore_barrier()

  def body(j, carry):
    pltpu.sync_copy(bm_hbm.at[s, j], idx_v)
    pltpu.sync_copy(
        x_hbm.at[pl.ds(s * (PCN * PCW) + j * PCW, PCW), pl.ds(col, D1)], rows_v)
    pltpu.sync_copy(rows_v, acc.at[idx_v], add=True)
    return carry

  lax.fori_loop(0, PCN, body, 0)
  plsc.subcore_barrier()
  pltpu.sync_copy(acc.at[pl.ds(s * GPT, GPT)],
                  out_hbm.at[pl.ds(s * GPT, GPT), pl.ds(col, D1)])


_TCR = 1000
_TCG = N // _TCR


def _dis(degp_ref):
  deg = degp_ref[0, :, 0:1] + degp_ref[1, :, 0:1] + 1.0
  return lax.rsqrt(deg)


def _row_spec(width):
  return pl.BlockSpec((_TCR, width), lambda i: (i, 0))


def _full_spec(shape):
  nd = len(shape)
  return pl.BlockSpec(shape, lambda i: (0,) * nd)


_DEG_SPEC = pl.BlockSpec((2, _TCR, DEG_W), lambda i: (0, i, 0))


def _tca_body(x_ref, w_ref, degp_ref, o_ref):
  h = jnp.dot(x_ref[...], w_ref[...], preferred_element_type=jnp.float32)
  o_ref[...] = h * _dis(degp_ref)


def _tc_g1(x, w0p, degp):
  return pl.pallas_call(
      _tca_body,
      out_shape=jax.ShapeDtypeStruct((N, D1), jnp.float32),
      grid=(_TCG,),
      in_specs=[_row_spec(128), _full_spec((128, D1)), _DEG_SPEC],
      out_specs=_row_spec(D1),
  )(x, w0p, degp)


def _tcb_body(p_ref, g_ref, degp_ref, b_ref, o_ref):
  dis = _dis(degp_ref)
  x1 = jnp.maximum(dis * (p_ref[0] + p_ref[1] + g_ref[...]) + b_ref[...], 0.0)
  o_ref[...] = dis * x1


def _tc_g2(p1, g1, degp, b0p):
  return pl.pallas_call(
      _tcb_body,
      out_shape=jax.ShapeDtypeStruct((N, D1), jnp.float32),
      grid=(_TCG,),
      in_specs=[pl.BlockSpec((2, _TCR, D1), lambda i: (0, i, 0)),
                _row_spec(D1), _DEG_SPEC, _full_spec((1, D1))],
      out_specs=_row_spec(D1),
  )(p1, g1, degp, b0p)


def _tcc_body(p_ref, g_ref, degp_ref, w1_ref, b1_ref, w2_ref, o_ref):
  dis = _dis(degp_ref)
  a2 = dis * (p_ref[0] + p_ref[1] + g_ref[...])
  x2 = jnp.maximum(
      jnp.dot(a2, w1_ref[...], preferred_element_type=jnp.float32) + b1_ref[...],
      0.0)
  h3 = jnp.dot(x2, w2_ref[...], preferred_element_type=jnp.float32)
  o_ref[...] = dis * h3


def _tc_g3(p2, g2, degp, w1p, b1, w2p):
  return pl.pallas_call(
      _tcc_body,
      out_shape=jax.ShapeDtypeStruct((N, D2), jnp.float32),
      grid=(_TCG,),
      in_specs=[pl.BlockSpec((2, _TCR, D1), lambda i: (0, i, 0)),
                _row_spec(D1), _DEG_SPEC,
                _full_spec((D1, 420)), _full_spec((1, 420)),
                _full_spec((420, D2))],
      out_specs=_row_spec(D2),
  )(p2, g2, degp, w1p, b1, w2p)


def _tcd_body(p_ref, g_ref, degp_ref, b_ref, w_ref, o_ref):
  dis = _dis(degp_ref)
  x3 = jnp.maximum(dis * (p_ref[...] + g_ref[...]) + b_ref[...], 0.0)
  h4 = jnp.dot(x3, w_ref[...], preferred_element_type=jnp.float32)
  o_ref[...] = dis * h4


def _tc_g4(p3, g3, degp, b2p, w3p):
  return pl.pallas_call(
      _tcd_body,
      out_shape=jax.ShapeDtypeStruct((N, D2), jnp.float32),
      grid=(_TCG,),
      in_specs=[_row_spec(D2), _row_spec(D2), _DEG_SPEC,
                _full_spec((1, D2)), _full_spec((D2, D2))],
      out_specs=_row_spec(D2),
  )(p3, g3, degp, b2p, w3p)


def _tce_body(p_ref, g_ref, degp_ref, b_ref, o_ref):
  dis = _dis(degp_ref)
  o_ref[...] = jnp.maximum(
      dis * (p_ref[...] + g_ref[...]) + b_ref[...], 0.0)


def _tc_x4(p4, g4, degp, b3p):
  return pl.pallas_call(
      _tce_body,
      out_shape=jax.ShapeDtypeStruct((N, D2), jnp.float32),
      grid=(_TCG,),
      in_specs=[_row_spec(D2), _row_spec(D2), _DEG_SPEC, _full_spec((1, D2))],
      out_specs=_row_spec(D2),
  )(p4, g4, degp, b3p)


def _tcf_body(q_ref, t_ref, mean_ref, std_ref,
              w0_ref, b0_ref, w1_ref, b1_ref, w2_ref, b2_ref, w3_ref, b3_ref,
              wa_ref, ba_ref, o_ref):
  pooled = jnp.maximum(q_ref[...], 0.0)
  m = jnp.maximum(
      jnp.dot(pooled, w0_ref[...], preferred_element_type=jnp.float32)
      + b0_ref[...], 0.0)
  m = jnp.maximum(
      jnp.dot(m, w1_ref[...], preferred_element_type=jnp.float32)
      + b1_ref[...], 0.0)
  m = jnp.maximum(
      jnp.dot(m, w2_ref[...], preferred_element_type=jnp.float32)
      + b2_ref[...], 0.0)
  m = jnp.maximum(
      jnp.dot(m, w3_ref[...], preferred_element_type=jnp.float32)
      + b3_ref[...], 0.0)
  coeff = jnp.dot(m, wa_ref[...], preferred_element_type=jnp.float32) + ba_ref[...]
  a = coeff[:, 0:1]
  b = coeff[:, 1:2]
  cc = coeff[:, 2:3]
  log_p = a - b / (t_ref[...] + cc)
  o_ref[...] = (log_p - mean_ref[...]) / std_ref[...]


def _tc_head(q, t, mean, std, w0p, b0, w1, b1, w2, b2, w3, b3, wap, bap):
  return pl.pallas_call(
      _tcf_body,
      out_shape=jax.ShapeDtypeStruct((G, 1), jnp.float32),
  )(q, t, mean, std, w0p, b0, w1, b1, w2, b2, w3, b3, wap, bap)


def kernel(x, edge_index, batch_mapping, temperature, mean, std,
           W_g0, b_g0, W_g1, b_g1, W_g2, b_g2, W_g3, b_g3,
           W_d0, b_d0, W_d1, b_d1, W_d2, b_d2, W_d3, b_d3,
           W_a, b_a):
  f32 = jnp.float32
  src_w = edge_index[0].reshape(NW, NCHUNK_W, CHUNK)
  dst_w = edge_index[1].reshape(NW, NCHUNK_W, CHUNK)
  src_s = edge_index[0].reshape(NS, NCHUNK_S, CHUNK)
  dst_s = edge_index[1].reshape(NS, NCHUNK_S, CHUNK)

  zeros_d1 = jnp.zeros((RPAD, D1), f32)
  zeros_dw = zeros_d1
  ones_dw = jnp.ones((CHUNK, DEG_W), f32)

  w0p = jnp.pad(W_g0, ((0, 0), (0, D1 - 100)))
  b0p = jnp.pad(b_g0, (0, D1 - 100)).reshape(1, D1)
  w1p = jnp.pad(W_g1, ((0, D1 - 100), (0, 0)))
  b1r = b_g1.reshape(1, 420)
  w2p = jnp.pad(W_g2, ((0, 0), (0, D2 - 140)))
  b2p = jnp.pad(b_g2, (0, D2 - 140)).reshape(1, D2)
  w3p = jnp.pad(W_g3, ((0, D2 - 140), (0, D2 - 140)))
  b3p = jnp.pad(b_g3, (0, D2 - 140)).reshape(1, D2)
  wd0p = jnp.pad(W_d0, ((0, D2 - 140), (0, 0)))

  degp = _degree_hist(dst_w, ones_dw, zeros_dw)

  g1 = _tc_g1(x, w0p, degp)
  p1 = _edge_scatter_128(g1, src_w, dst_w, zeros_d1)
  g2 = _tc_g2(p1, g1, degp, b0p)
  p2 = _edge_scatter_128(g2, src_w, dst_w, zeros_d1)
  g3 = _tc_g3(p2, g2, degp, w1p, b1r, w2p)
  p3 = _edge_scatter_256(g3, src_s, dst_s, zeros_d1)
  g4 = _tc_g4(p3, g3, degp, b2p, w3p)
  p4 = _edge_scatter_256(g4, src_s, dst_s, zeros_d1)
  x4 = _tc_x4(p4, g4, degp, b3p)

  x4p = jnp.pad(x4, ((0, RPAD - N), (0, 0)))
  bmp = jnp.pad(batch_mapping, (0, RPAD - N)).reshape(NS, PCN, PCW)
  q = _pool_scatter(x4p, bmp, zeros_d1)

  out = _tc_head(q, temperature.reshape(G, 1), mean.reshape(1, 1),
                 std.reshape(1, 1), wd0p, b_d0.reshape(1, 260),
                 W_d1, b_d1.reshape(1, 60), W_d2, b_d2.reshape(1, 180),
                 W_d3, b_d3.reshape(1, 100), W_a, b_a.reshape(1, 3))
  return out.reshape(G)


# speedup vs baseline: 18.2475x; 1.7119x over previous
import functools

import jax
import jax.numpy as jnp
from jax import lax
from jax.experimental import pallas as pl
from jax.experimental.pallas import tpu as pltpu
from jax.experimental.pallas import tpu_sc as plsc

N = 10000
E = 320000
G = 256

NC = 2
NS = 16
NW = NC * NS
CHUNK = 125
NCHUNK_W = E // NW // CHUNK
NCHUNK_S = E // NS // CHUNK
RPAD = 10240
RPW = RPAD // NS
DEG_W = 128

D1 = 128
D2 = 256

PCW = 128
PCN = RPAD // NS // PCW
GPT = G // NS

_SC_MESH = plsc.VectorSubcoreMesh(core_axis_name="c", subcore_axis_name="s")


def _pipelined_edge_loop(nchunk, src_hbm, dst_hbm, row_at, gather_src,
                         acc, src_v, dst_v, rows_v, sems):
  sem_si, sem_di, sem_g, sem_sc = sems

  def idx_copies(j, slot):
    return (pltpu.make_async_copy(src_hbm.at[row_at(j)], src_v.at[slot],
                                  sem_si.at[slot]),
            pltpu.make_async_copy(dst_hbm.at[row_at(j)], dst_v.at[slot],
                                  sem_di.at[slot]))

  def gather_copy(j, b4, b2):
    return pltpu.make_async_copy(gather_src(src_v.at[b4]), rows_v.at[b2],
                                 sem_g.at[b2])

  def scatter_copy(b4, b2):
    return pltpu.make_async_copy(rows_v.at[b2], acc.at[dst_v.at[b4]],
                                 sem_sc.at[b2])

  for cp in idx_copies(0, 0) + idx_copies(1, 1):
    cp.start()

  def body(jo, carry):
    for k in range(4):
      j = jo * 4 + k
      b2 = k % 2
      b4 = k
      b4n = (k + 2) % 4
      for cp in idx_copies(j, b4):
        cp.wait()

      @pl.when(j >= 2)
      def _():
        scatter_copy(b4n, b2).wait()

      gather_copy(j, b4, b2).start()

      @pl.when(j + 2 < nchunk)
      def _():
        for cp in idx_copies(j + 2, b4n):
          cp.start()

      gather_copy(j, b4, b2).wait()
      scatter_copy(b4, b2).start(add=True)
    return carry

  lax.fori_loop(0, nchunk // 4, body, 0)
  scatter_copy((nchunk - 2) % 4, 0).wait()
  scatter_copy((nchunk - 1) % 4, 1).wait()


_EDGE_SCRATCH = [
    pltpu.VMEM((4, CHUNK), jnp.int32),
    pltpu.VMEM((4, CHUNK), jnp.int32),
    pltpu.VMEM((2, CHUNK, D1), jnp.float32),
    pltpu.VMEM_SHARED((RPAD, D1), jnp.float32),
    pltpu.SemaphoreType.DMA((4,)),
    pltpu.SemaphoreType.DMA((4,)),
    pltpu.SemaphoreType.DMA((2,)),
    pltpu.SemaphoreType.DMA((2,)),
]


@functools.partial(
    pl.kernel,
    mesh=_SC_MESH,
    out_type=jax.ShapeDtypeStruct((NC, RPAD, D1), jnp.float32),
    scratch_types=_EDGE_SCRATCH,
)
def _edge_scatter_128(g_hbm, src_hbm, dst_hbm, zeros_hbm, out_hbm,
                      src_v, dst_v, rows_v, acc, sem_si, sem_di, sem_g, sem_sc):
  c = lax.axis_index("c")
  s = lax.axis_index("s")
  wid = c * NS + s
  pltpu.sync_copy(zeros_hbm.at[pl.ds(s * RPW, RPW)], acc.at[pl.ds(s * RPW, RPW)])
  plsc.subcore_barrier()
  _pipelined_edge_loop(
      NCHUNK_W, src_hbm, dst_hbm,
      row_at=lambda j: (wid, j),
      gather_src=lambda idx: g_hbm.at[idx],
      acc=acc, src_v=src_v, dst_v=dst_v, rows_v=rows_v,
      sems=(sem_si, sem_di, sem_g, sem_sc))
  plsc.subcore_barrier()
  pltpu.sync_copy(acc.at[pl.ds(s * RPW, RPW)], out_hbm.at[c, pl.ds(s * RPW, RPW)])


@functools.partial(
    pl.kernel,
    mesh=_SC_MESH,
    out_type=jax.ShapeDtypeStruct((RPAD, D2), jnp.float32),
    scratch_types=_EDGE_SCRATCH,
)
def _edge_scatter_256(g_hbm, src_hbm, dst_hbm, zeros_hbm, out_hbm,
                      src_v, dst_v, rows_v, acc, sem_si, sem_di, sem_g, sem_sc):
  c = lax.axis_index("c")
  s = lax.axis_index("s")
  col = pl.multiple_of(c * D1, D1)
  pltpu.sync_copy(zeros_hbm.at[pl.ds(s * RPW, RPW)], acc.at[pl.ds(s * RPW, RPW)])
  plsc.subcore_barrier()
  _pipelined_edge_loop(
      NCHUNK_S, src_hbm, dst_hbm,
      row_at=lambda j: (s, j),
      gather_src=lambda idx: g_hbm.at[idx, pl.ds(col, D1)],
      acc=acc, src_v=src_v, dst_v=dst_v, rows_v=rows_v,
      sems=(sem_si, sem_di, sem_g, sem_sc))
  plsc.subcore_barrier()
  pltpu.sync_copy(acc.at[pl.ds(s * RPW, RPW)],
                  out_hbm.at[pl.ds(s * RPW, RPW), pl.ds(col, D1)])


@functools.partial(
    pl.kernel,
    mesh=_SC_MESH,
    out_type=jax.ShapeDtypeStruct((NC, RPAD, DEG_W), jnp.float32),
    scratch_types=[
        pltpu.VMEM((CHUNK,), jnp.int32),
        pltpu.VMEM((CHUNK, DEG_W), jnp.float32),
        pltpu.VMEM_SHARED((RPAD, DEG_W), jnp.float32),
    ],
)
def _degree_hist(dst_hbm, ones_hbm, zeros_hbm, out_hbm, dst_v, ones_v, acc):
  c = lax.axis_index("c")
  s = lax.axis_index("s")
  wid = c * NS + s
  pltpu.sync_copy(ones_hbm, ones_v)
  pltpu.sync_copy(zeros_hbm.at[pl.ds(s * RPW, RPW)], acc.at[pl.ds(s * RPW, RPW)])
  plsc.subcore_barrier()

  def body(j, carry):
    pltpu.sync_copy(dst_hbm.at[wid, j], dst_v)
    pltpu.sync_copy(ones_v, acc.at[dst_v], add=True)
    return carry

  lax.fori_loop(0, NCHUNK_W, body, 0)
  plsc.subcore_barrier()
  pltpu.sync_copy(acc.at[pl.ds(s * RPW, RPW)], out_hbm.at[c, pl.ds(s * RPW, RPW)])


@functools.partial(
    pl.kernel,
    mesh=_SC_MESH,
    out_type=jax.ShapeDtypeStruct((G, D2), jnp.float32),
    scratch_types=[
        pltpu.VMEM((PCW,), jnp.int32),
        pltpu.VMEM((PCW, D1), jnp.float32),
        pltpu.VMEM_SHARED((G, D1), jnp.float32),
    ],
)
def _pool_scatter(x_hbm, bm_hbm, zeros_hbm, out_hbm, idx_v, rows_v, acc):
  c = lax.axis_index("c")
  s = lax.axis_index("s")
  col = pl.multiple_of(c * D1, D1)
  pltpu.sync_copy(zeros_hbm.at[pl.ds(s * GPT, GPT)], acc.at[pl.ds(s * GPT, GPT)])
  plsc.subcore_barrier()

  def body(j, carry):
    pltpu.sync_copy(bm_hbm.at[s, j], idx_v)
    pltpu.sync_copy(
        x_hbm.at[pl.ds(s * (PCN * PCW) + j * PCW, PCW), pl.ds(col, D1)], rows_v)
    pltpu.sync_copy(rows_v, acc.at[idx_v], add=True)
    return carry

  lax.fori_loop(0, PCN, body, 0)
  plsc.subcore_barrier()
  pltpu.sync_copy(acc.at[pl.ds(s * GPT, GPT)],
                  out_hbm.at[pl.ds(s * GPT, GPT), pl.ds(col, D1)])


_TCR = 1000
_TCG = N // _TCR


def _dis(degp_ref):
  deg = degp_ref[0, :, 0:1] + degp_ref[1, :, 0:1] + 1.0
  return lax.rsqrt(deg)


def _row_spec(width):
  return pl.BlockSpec((_TCR, width), lambda i: (i, 0))


def _full_spec(shape):
  nd = len(shape)
  return pl.BlockSpec(shape, lambda i: (0,) * nd)


_DEG_SPEC = pl.BlockSpec((2, _TCR, DEG_W), lambda i: (0, i, 0))


def _tca_body(x_ref, w_ref, degp_ref, o_ref):
  h = jnp.dot(x_ref[...], w_ref[...], preferred_element_type=jnp.float32)
  o_ref[...] = h * _dis(degp_ref)


def _tc_g1(x, w0p, degp):
  return pl.pallas_call(
      _tca_body,
      out_shape=jax.ShapeDtypeStruct((N, D1), jnp.float32),
      grid=(_TCG,),
      in_specs=[_row_spec(128), _full_spec((128, D1)), _DEG_SPEC],
      out_specs=_row_spec(D1),
  )(x, w0p, degp)


def _tcb_body(p_ref, g_ref, degp_ref, b_ref, o_ref):
  dis = _dis(degp_ref)
  x1 = jnp.maximum(dis * (p_ref[0] + p_ref[1] + g_ref[...]) + b_ref[...], 0.0)
  o_ref[...] = dis * x1


def _tc_g2(p1, g1, degp, b0p):
  return pl.pallas_call(
      _tcb_body,
      out_shape=jax.ShapeDtypeStruct((N, D1), jnp.float32),
      grid=(_TCG,),
      in_specs=[pl.BlockSpec((2, _TCR, D1), lambda i: (0, i, 0)),
                _row_spec(D1), _DEG_SPEC, _full_spec((1, D1))],
      out_specs=_row_spec(D1),
  )(p1, g1, degp, b0p)


def _tcc_body(p_ref, g_ref, degp_ref, w1_ref, b1_ref, w2_ref, o_ref):
  dis = _dis(degp_ref)
  a2 = dis * (p_ref[0] + p_ref[1] + g_ref[...])
  x2 = jnp.maximum(
      jnp.dot(a2, w1_ref[...], preferred_element_type=jnp.float32) + b1_ref[...],
      0.0)
  h3 = jnp.dot(x2, w2_ref[...], preferred_element_type=jnp.float32)
  o_ref[...] = dis * h3


def _tc_g3(p2, g2, degp, w1p, b1, w2p):
  return pl.pallas_call(
      _tcc_body,
      out_shape=jax.ShapeDtypeStruct((N, D2), jnp.float32),
      grid=(_TCG,),
      in_specs=[pl.BlockSpec((2, _TCR, D1), lambda i: (0, i, 0)),
                _row_spec(D1), _DEG_SPEC,
                _full_spec((D1, 420)), _full_spec((1, 420)),
                _full_spec((420, D2))],
      out_specs=_row_spec(D2),
  )(p2, g2, degp, w1p, b1, w2p)


def _tcd_body(p_ref, g_ref, degp_ref, b_ref, w_ref, o_ref):
  dis = _dis(degp_ref)
  x3 = jnp.maximum(dis * (p_ref[...] + g_ref[...]) + b_ref[...], 0.0)
  h4 = jnp.dot(x3, w_ref[...], preferred_element_type=jnp.float32)
  o_ref[...] = dis * h4


def _tc_g4(p3, g3, degp, b2p, w3p):
  return pl.pallas_call(
      _tcd_body,
      out_shape=jax.ShapeDtypeStruct((N, D2), jnp.float32),
      grid=(_TCG,),
      in_specs=[_row_spec(D2), _row_spec(D2), _DEG_SPEC,
                _full_spec((1, D2)), _full_spec((D2, D2))],
      out_specs=_row_spec(D2),
  )(p3, g3, degp, b2p, w3p)


def _tce_body(p_ref, g_ref, degp_ref, b_ref, o_ref):
  dis = _dis(degp_ref)
  o_ref[...] = jnp.maximum(
      dis * (p_ref[...] + g_ref[...]) + b_ref[...], 0.0)


def _tc_x4(p4, g4, degp, b3p):
  return pl.pallas_call(
      _tce_body,
      out_shape=jax.ShapeDtypeStruct((N, D2), jnp.float32),
      grid=(_TCG,),
      in_specs=[_row_spec(D2), _row_spec(D2), _DEG_SPEC, _full_spec((1, D2))],
      out_specs=_row_spec(D2),
  )(p4, g4, degp, b3p)


def _tcf_body(q_ref, t_ref, mean_ref, std_ref,
              w0_ref, b0_ref, w1_ref, b1_ref, w2_ref, b2_ref, w3_ref, b3_ref,
              wa_ref, ba_ref, o_ref):
  pooled = jnp.maximum(q_ref[...], 0.0)
  m = jnp.maximum(
      jnp.dot(pooled, w0_ref[...], preferred_element_type=jnp.float32)
      + b0_ref[...], 0.0)
  m = jnp.maximum(
      jnp.dot(m, w1_ref[...], preferred_element_type=jnp.float32)
      + b1_ref[...], 0.0)
  m = jnp.maximum(
      jnp.dot(m, w2_ref[...], preferred_element_type=jnp.float32)
      + b2_ref[...], 0.0)
  m = jnp.maximum(
      jnp.dot(m, w3_ref[...], preferred_element_type=jnp.float32)
      + b3_ref[...], 0.0)
  coeff = jnp.dot(m, wa_ref[...], preferred_element_type=jnp.float32) + ba_ref[...]
  a = coeff[:, 0:1]
  b = coeff[:, 1:2]
  cc = coeff[:, 2:3]
  log_p = a - b / (t_ref[...] + cc)
  o_ref[...] = (log_p - mean_ref[...]) / std_ref[...]


def _tc_head(q, t, mean, std, w0p, b0, w1, b1, w2, b2, w3, b3, wap, bap):
  return pl.pallas_call(
      _tcf_body,
      out_shape=jax.ShapeDtypeStruct((G, 1), jnp.float32),
  )(q, t, mean, std, w0p, b0, w1, b1, w2, b2, w3, b3, wap, bap)


def kernel(x, edge_index, batch_mapping, temperature, mean, std,
           W_g0, b_g0, W_g1, b_g1, W_g2, b_g2, W_g3, b_g3,
           W_d0, b_d0, W_d1, b_d1, W_d2, b_d2, W_d3, b_d3,
           W_a, b_a):
  f32 = jnp.float32
  src_w = edge_index[0].reshape(NW, NCHUNK_W, CHUNK)
  dst_w = edge_index[1].reshape(NW, NCHUNK_W, CHUNK)
  src_s = edge_index[0].reshape(NS, NCHUNK_S, CHUNK)
  dst_s = edge_index[1].reshape(NS, NCHUNK_S, CHUNK)

  zeros_d1 = jnp.zeros((RPAD, D1), f32)
  zeros_dw = zeros_d1
  ones_dw = jnp.ones((CHUNK, DEG_W), f32)

  w0p = jnp.pad(W_g0, ((0, 0), (0, D1 - 100)))
  b0p = jnp.pad(b_g0, (0, D1 - 100)).reshape(1, D1)
  w1p = jnp.pad(W_g1, ((0, D1 - 100), (0, 0)))
  b1r = b_g1.reshape(1, 420)
  w2p = jnp.pad(W_g2, ((0, 0), (0, D2 - 140)))
  b2p = jnp.pad(b_g2, (0, D2 - 140)).reshape(1, D2)
  w3p = jnp.pad(W_g3, ((0, D2 - 140), (0, D2 - 140)))
  b3p = jnp.pad(b_g3, (0, D2 - 140)).reshape(1, D2)
  wd0p = jnp.pad(W_d0, ((0, D2 - 140), (0, 0)))

  degp = _degree_hist(dst_w, ones_dw, zeros_dw)

  g1 = _tc_g1(x, w0p, degp)
  p1 = _edge_scatter_128(g1, src_w, dst_w, zeros_d1)
  g2 = _tc_g2(p1, g1, degp, b0p)
  p2 = _edge_scatter_128(g2, src_w, dst_w, zeros_d1)
  g3 = _tc_g3(p2, g2, degp, w1p, b1r, w2p)
  p3 = _edge_scatter_256(g3, src_s, dst_s, zeros_d1)
  g4 = _tc_g4(p3, g3, degp, b2p, w3p)
  p4 = _edge_scatter_256(g4, src_s, dst_s, zeros_d1)
  x4 = _tc_x4(p4, g4, degp, b3p)

  x4p = jnp.pad(x4, ((0, RPAD - N), (0, 0)))
  bmp = jnp.pad(batch_mapping, (0, RPAD - N)).reshape(NS, PCN, PCW)
  q = _pool_scatter(x4p, bmp, zeros_d1)

  out = _tc_head(q, temperature.reshape(G, 1), mean.reshape(1, 1),
                 std.reshape(1, 1), wd0p, b_d0.reshape(1, 260),
                 W_d1, b_d1.reshape(1, 60), W_d2, b_d2.reshape(1, 180),
                 W_d3, b_d3.reshape(1, 100), W_a, b_a.reshape(1, 3))
  return out.reshape(G)


# speedup vs baseline: 21.4430x; 1.1751x over previous
import functools

import jax
import jax.numpy as jnp
from jax import lax
from jax.experimental import pallas as pl
from jax.experimental.pallas import tpu as pltpu
from jax.experimental.pallas import tpu_sc as plsc

N = 10000
E = 320000
G = 256

NC = 2
NS = 16
NW = NC * NS
CHUNK = 125
NCHUNK_W = E // NW // CHUNK
NCHUNK_S = E // NS // CHUNK
RPAD = 10240
RPW = RPAD // NS
DEG_W = 128

D1 = 128
D2 = 256

PCW = 128
PCN = RPAD // NS // PCW
GPT = G // NS

_SC_MESH = plsc.VectorSubcoreMesh(core_axis_name="c", subcore_axis_name="s")


def _pipelined_edge_loop(nchunk, src_hbm, dst_hbm, row_at, gather_src,
                         acc, src_v, dst_v, rows_v, sems):
  sem_si, sem_di, sem_g, sem_sc = sems

  def idx_copies(j, slot):
    return (pltpu.make_async_copy(src_hbm.at[row_at(j)], src_v.at[slot],
                                  sem_si.at[slot]),
            pltpu.make_async_copy(dst_hbm.at[row_at(j)], dst_v.at[slot],
                                  sem_di.at[slot]))

  def gather_copy(j, b4, b2):
    return pltpu.make_async_copy(gather_src(src_v.at[b4]), rows_v.at[b2],
                                 sem_g.at[b2])

  def scatter_copy(b4, b2):
    return pltpu.make_async_copy(rows_v.at[b2], acc.at[dst_v.at[b4]],
                                 sem_sc.at[b2])

  for cp in idx_copies(0, 0) + idx_copies(1, 1):
    cp.start()
  for cp in idx_copies(0, 0):
    cp.wait()
  gather_copy(0, 0, 0).start()

  def body(jo, carry):
    for k in range(4):
      j = jo * 4 + k
      b2 = k % 2
      b4 = k
      n2 = (k + 1) % 2
      n4 = (k + 1) % 4
      p4 = (k + 3) % 4
      f4 = (k + 2) % 4

      @pl.when(j + 1 < nchunk)
      def _():
        for cp in idx_copies(j + 1, n4):
          cp.wait()

      @pl.when(j >= 1)
      def _():
        scatter_copy(p4, n2).wait()

      @pl.when(j + 1 < nchunk)
      def _():
        gather_copy(j + 1, n4, n2).start()

      @pl.when(j + 2 < nchunk)
      def _():
        for cp in idx_copies(j + 2, f4):
          cp.start()

      gather_copy(j, b4, b2).wait()
      scatter_copy(b4, b2).start(add=True)
    return carry

  lax.fori_loop(0, nchunk // 4, body, 0)
  scatter_copy((nchunk - 1) % 4, (nchunk - 1) % 2).wait()


_EDGE_SCRATCH = [
    pltpu.VMEM((4, CHUNK), jnp.int32),
    pltpu.VMEM((4, CHUNK), jnp.int32),
    pltpu.VMEM((2, CHUNK, D1), jnp.float32),
    pltpu.VMEM_SHARED((RPAD, D1), jnp.float32),
    pltpu.SemaphoreType.DMA((4,)),
    pltpu.SemaphoreType.DMA((4,)),
    pltpu.SemaphoreType.DMA((2,)),
    pltpu.SemaphoreType.DMA((2,)),
]


@functools.partial(
    pl.kernel,
    mesh=_SC_MESH,
    out_type=jax.ShapeDtypeStruct((NC, RPAD, D1), jnp.float32),
    scratch_types=_EDGE_SCRATCH,
)
def _edge_scatter_128(g_hbm, src_hbm, dst_hbm, zeros_hbm, out_hbm,
                      src_v, dst_v, rows_v, acc, sem_si, sem_di, sem_g, sem_sc):
  c = lax.axis_index("c")
  s = lax.axis_index("s")
  wid = c * NS + s
  pltpu.sync_copy(zeros_hbm.at[pl.ds(s * RPW, RPW)], acc.at[pl.ds(s * RPW, RPW)])
  plsc.subcore_barrier()
  _pipelined_edge_loop(
      NCHUNK_W, src_hbm, dst_hbm,
      row_at=lambda j: (wid, j),
      gather_src=lambda idx: g_hbm.at[idx],
      acc=acc, src_v=src_v, dst_v=dst_v, rows_v=rows_v,
      sems=(sem_si, sem_di, sem_g, sem_sc))
  plsc.subcore_barrier()
  pltpu.sync_copy(acc.at[pl.ds(s * RPW, RPW)], out_hbm.at[c, pl.ds(s * RPW, RPW)])


@functools.partial(
    pl.kernel,
    mesh=_SC_MESH,
    out_type=jax.ShapeDtypeStruct((RPAD, D2), jnp.float32),
    scratch_types=_EDGE_SCRATCH,
)
def _edge_scatter_256(g_hbm, src_hbm, dst_hbm, zeros_hbm, out_hbm,
                      src_v, dst_v, rows_v, acc, sem_si, sem_di, sem_g, sem_sc):
  c = lax.axis_index("c")
  s = lax.axis_index("s")
  col = pl.multiple_of(c * D1, D1)
  pltpu.sync_copy(zeros_hbm.at[pl.ds(s * RPW, RPW)], acc.at[pl.ds(s * RPW, RPW)])
  plsc.subcore_barrier()
  _pipelined_edge_loop(
      NCHUNK_S, src_hbm, dst_hbm,
      row_at=lambda j: (s, j),
      gather_src=lambda idx: g_hbm.at[idx, pl.ds(col, D1)],
      acc=acc, src_v=src_v, dst_v=dst_v, rows_v=rows_v,
      sems=(sem_si, sem_di, sem_g, sem_sc))
  plsc.subcore_barrier()
  pltpu.sync_copy(acc.at[pl.ds(s * RPW, RPW)],
                  out_hbm.at[pl.ds(s * RPW, RPW), pl.ds(col, D1)])


@functools.partial(
    pl.kernel,
    mesh=_SC_MESH,
    out_type=jax.ShapeDtypeStruct((NC, RPAD, DEG_W), jnp.float32),
    scratch_types=[
        pltpu.VMEM((4, CHUNK), jnp.int32),
        pltpu.VMEM((CHUNK, DEG_W), jnp.float32),
        pltpu.VMEM_SHARED((RPAD, DEG_W), jnp.float32),
        pltpu.SemaphoreType.DMA((4,)),
        pltpu.SemaphoreType.DMA((2,)),
    ],
)
def _degree_hist(dst_hbm, ones_hbm, zeros_hbm, out_hbm, dst_v, ones_v, acc,
                 sem_di, sem_sc):
  c = lax.axis_index("c")
  s = lax.axis_index("s")
  wid = c * NS + s
  pltpu.sync_copy(ones_hbm, ones_v)
  pltpu.sync_copy(zeros_hbm.at[pl.ds(s * RPW, RPW)], acc.at[pl.ds(s * RPW, RPW)])
  plsc.subcore_barrier()

  def idx_copy(j, slot):
    return pltpu.make_async_copy(dst_hbm.at[wid, j], dst_v.at[slot],
                                 sem_di.at[slot])

  def scatter_copy(b4, b2):
    return pltpu.make_async_copy(ones_v, acc.at[dst_v.at[b4]], sem_sc.at[b2])

  idx_copy(0, 0).start()
  idx_copy(1, 1).start()

  def body(jo, carry):
    for k in range(4):
      j = jo * 4 + k
      b2 = k % 2
      b4 = k
      f4 = (k + 2) % 4
      idx_copy(j, b4).wait()

      @pl.when(j >= 2)
      def _():
        scatter_copy(f4, b2).wait()

      @pl.when(j + 2 < NCHUNK_W)
      def _():
        idx_copy(j + 2, f4).start()

      scatter_copy(b4, b2).start(add=True)
    return carry

  lax.fori_loop(0, NCHUNK_W // 4, body, 0)
  scatter_copy((NCHUNK_W - 2) % 4, 0).wait()
  scatter_copy((NCHUNK_W - 1) % 4, 1).wait()
  plsc.subcore_barrier()
  pltpu.sync_copy(acc.at[pl.ds(s * RPW, RPW)], out_hbm.at[c, pl.ds(s * RPW, RPW)])


@functools.partial(
    pl.kernel,
    mesh=_SC_MESH,
    out_type=jax.ShapeDtypeStruct((G, D2), jnp.float32),
    scratch_types=[
        pltpu.VMEM((PCW,), jnp.int32),
        pltpu.VMEM((PCW, D1), jnp.float32),
        pltpu.VMEM_SHARED((G, D1), jnp.float32),
    ],
)
def _pool_scatter(x_hbm, bm_hbm, zeros_hbm, out_hbm, idx_v, rows_v, acc):
  c = lax.axis_index("c")
  s = lax.axis_index("s")
  col = pl.multiple_of(c * D1, D1)
  pltpu.sync_copy(zeros_hbm.at[pl.ds(s * GPT, GPT)], acc.at[pl.ds(s * GPT, GPT)])
  plsc.subcore_barrier()

  def body(j, carry):
    pltpu.sync_copy(bm_hbm.at[s, j], idx_v)
    pltpu.sync_copy(
        x_hbm.at[pl.ds(s * (PCN * PCW) + j * PCW, PCW), pl.ds(col, D1)], rows_v)
    pltpu.sync_copy(rows_v, acc.at[idx_v], add=True)
    return carry

  lax.fori_loop(0, PCN, body, 0)
  plsc.subcore_barrier()
  pltpu.sync_copy(acc.at[pl.ds(s * GPT, GPT)],
                  out_hbm.at[pl.ds(s * GPT, GPT), pl.ds(col, D1)])


_TCR = 1000
_TCG = N // _TCR


def _dis(degp_ref):
  deg = degp_ref[0, :, 0:1] + degp_ref[1, :, 0:1] + 1.0
  return lax.rsqrt(deg)


def _row_spec(width):
  return pl.BlockSpec((_TCR, width), lambda i: (i, 0))


def _full_spec(shape):
  nd = len(shape)
  return pl.BlockSpec(shape, lambda i: (0,) * nd)


_DEG_SPEC = pl.BlockSpec((2, _TCR, DEG_W), lambda i: (0, i, 0))


def _tca_body(x_ref, w_ref, degp_ref, o_ref):
  h = jnp.dot(x_ref[...], w_ref[...], preferred_element_type=jnp.float32)
  o_ref[...] = h * _dis(degp_ref)


def _tc_g1(x, w0p, degp):
  return pl.pallas_call(
      _tca_body,
      out_shape=jax.ShapeDtypeStruct((N, D1), jnp.float32),
      grid=(_TCG,),
      in_specs=[_row_spec(128), _full_spec((128, D1)), _DEG_SPEC],
      out_specs=_row_spec(D1),
  )(x, w0p, degp)


def _tcb_body(p_ref, g_ref, degp_ref, b_ref, o_ref):
  dis = _dis(degp_ref)
  x1 = jnp.maximum(dis * (p_ref[0] + p_ref[1] + g_ref[...]) + b_ref[...], 0.0)
  o_ref[...] = dis * x1


def _tc_g2(p1, g1, degp, b0p):
  return pl.pallas_call(
      _tcb_body,
      out_shape=jax.ShapeDtypeStruct((N, D1), jnp.float32),
      grid=(_TCG,),
      in_specs=[pl.BlockSpec((2, _TCR, D1), lambda i: (0, i, 0)),
                _row_spec(D1), _DEG_SPEC, _full_spec((1, D1))],
      out_specs=_row_spec(D1),
  )(p1, g1, degp, b0p)


def _tcc_body(p_ref, g_ref, degp_ref, w1_ref, b1_ref, w2_ref, o_ref):
  dis = _dis(degp_ref)
  a2 = dis * (p_ref[0] + p_ref[1] + g_ref[...])
  x2 = jnp.maximum(
      jnp.dot(a2, w1_ref[...], preferred_element_type=jnp.float32) + b1_ref[...],
      0.0)
  h3 = jnp.dot(x2, w2_ref[...], preferred_element_type=jnp.float32)
  o_ref[...] = dis * h3


def _tc_g3(p2, g2, degp, w1p, b1, w2p):
  return pl.pallas_call(
      _tcc_body,
      out_shape=jax.ShapeDtypeStruct((N, D2), jnp.float32),
      grid=(_TCG,),
      in_specs=[pl.BlockSpec((2, _TCR, D1), lambda i: (0, i, 0)),
                _row_spec(D1), _DEG_SPEC,
                _full_spec((D1, 420)), _full_spec((1, 420)),
                _full_spec((420, D2))],
      out_specs=_row_spec(D2),
  )(p2, g2, degp, w1p, b1, w2p)


def _tcd_body(p_ref, g_ref, degp_ref, b_ref, w_ref, o_ref):
  dis = _dis(degp_ref)
  x3 = jnp.maximum(dis * (p_ref[...] + g_ref[...]) + b_ref[...], 0.0)
  h4 = jnp.dot(x3, w_ref[...], preferred_element_type=jnp.float32)
  o_ref[...] = dis * h4


def _tc_g4(p3, g3, degp, b2p, w3p):
  return pl.pallas_call(
      _tcd_body,
      out_shape=jax.ShapeDtypeStruct((N, D2), jnp.float32),
      grid=(_TCG,),
      in_specs=[_row_spec(D2), _row_spec(D2), _DEG_SPEC,
                _full_spec((1, D2)), _full_spec((D2, D2))],
      out_specs=_row_spec(D2),
  )(p3, g3, degp, b2p, w3p)


def _tce_body(p_ref, g_ref, degp_ref, b_ref, o_ref):
  dis = _dis(degp_ref)
  o_ref[...] = jnp.maximum(
      dis * (p_ref[...] + g_ref[...]) + b_ref[...], 0.0)


def _tc_x4(p4, g4, degp, b3p):
  return pl.pallas_call(
      _tce_body,
      out_shape=jax.ShapeDtypeStruct((N, D2), jnp.float32),
      grid=(_TCG,),
      in_specs=[_row_spec(D2), _row_spec(D2), _DEG_SPEC, _full_spec((1, D2))],
      out_specs=_row_spec(D2),
  )(p4, g4, degp, b3p)


def _tcf_body(q_ref, t_ref, mean_ref, std_ref,
              w0_ref, b0_ref, w1_ref, b1_ref, w2_ref, b2_ref, w3_ref, b3_ref,
              wa_ref, ba_ref, o_ref):
  pooled = jnp.maximum(q_ref[...], 0.0)
  m = jnp.maximum(
      jnp.dot(pooled, w0_ref[...], preferred_element_type=jnp.float32)
      + b0_ref[...], 0.0)
  m = jnp.maximum(
      jnp.dot(m, w1_ref[...], preferred_element_type=jnp.float32)
      + b1_ref[...], 0.0)
  m = jnp.maximum(
      jnp.dot(m, w2_ref[...], preferred_element_type=jnp.float32)
      + b2_ref[...], 0.0)
  m = jnp.maximum(
      jnp.dot(m, w3_ref[...], preferred_element_type=jnp.float32)
      + b3_ref[...], 0.0)
  coeff = jnp.dot(m, wa_ref[...], preferred_element_type=jnp.float32) + ba_ref[...]
  a = coeff[:, 0:1]
  b = coeff[:, 1:2]
  cc = coeff[:, 2:3]
  log_p = a - b / (t_ref[...] + cc)
  o_ref[...] = (log_p - mean_ref[...]) / std_ref[...]


def _tc_head(q, t, mean, std, w0p, b0, w1, b1, w2, b2, w3, b3, wap, bap):
  return pl.pallas_call(
      _tcf_body,
      out_shape=jax.ShapeDtypeStruct((G, 1), jnp.float32),
  )(q, t, mean, std, w0p, b0, w1, b1, w2, b2, w3, b3, wap, bap)


def kernel(x, edge_index, batch_mapping, temperature, mean, std,
           W_g0, b_g0, W_g1, b_g1, W_g2, b_g2, W_g3, b_g3,
           W_d0, b_d0, W_d1, b_d1, W_d2, b_d2, W_d3, b_d3,
           W_a, b_a):
  f32 = jnp.float32
  src_w = edge_index[0].reshape(NW, NCHUNK_W, CHUNK)
  dst_w = edge_index[1].reshape(NW, NCHUNK_W, CHUNK)
  src_s = edge_index[0].reshape(NS, NCHUNK_S, CHUNK)
  dst_s = edge_index[1].reshape(NS, NCHUNK_S, CHUNK)

  zeros_d1 = jnp.zeros((RPAD, D1), f32)
  zeros_dw = zeros_d1
  ones_dw = jnp.ones((CHUNK, DEG_W), f32)

  w0p = jnp.pad(W_g0, ((0, 0), (0, D1 - 100)))
  b0p = jnp.pad(b_g0, (0, D1 - 100)).reshape(1, D1)
  w1p = jnp.pad(W_g1, ((0, D1 - 100), (0, 0)))
  b1r = b_g1.reshape(1, 420)
  w2p = jnp.pad(W_g2, ((0, 0), (0, D2 - 140)))
  b2p = jnp.pad(b_g2, (0, D2 - 140)).reshape(1, D2)
  w3p = jnp.pad(W_g3, ((0, D2 - 140), (0, D2 - 140)))
  b3p = jnp.pad(b_g3, (0, D2 - 140)).reshape(1, D2)
  wd0p = jnp.pad(W_d0, ((0, D2 - 140), (0, 0)))

  degp = _degree_hist(dst_w, ones_dw, zeros_dw)

  g1 = _tc_g1(x, w0p, degp)
  p1 = _edge_scatter_128(g1, src_w, dst_w, zeros_d1)
  g2 = _tc_g2(p1, g1, degp, b0p)
  p2 = _edge_scatter_128(g2, src_w, dst_w, zeros_d1)
  g3 = _tc_g3(p2, g2, degp, w1p, b1r, w2p)
  p3 = _edge_scatter_256(g3, src_s, dst_s, zeros_d1)
  g4 = _tc_g4(p3, g3, degp, b2p, w3p)
  p4 = _edge_scatter_256(g4, src_s, dst_s, zeros_d1)
  x4 = _tc_x4(p4, g4, degp, b3p)

  x4p = jnp.pad(x4, ((0, RPAD - N), (0, 0)))
  bmp = jnp.pad(batch_mapping, (0, RPAD - N)).reshape(NS, PCN, PCW)
  q = _pool_scatter(x4p, bmp, zeros_d1)

  out = _tc_head(q, temperature.reshape(G, 1), mean.reshape(1, 1),
                 std.reshape(1, 1), wd0p, b_d0.reshape(1, 260),
                 W_d1, b_d1.reshape(1, 60), W_d2, b_d2.reshape(1, 180),
                 W_d3, b_d3.reshape(1, 100), W_a, b_a.reshape(1, 3))
  return out.reshape(G)
